# baseline (speedup 1.0000x reference)
import functools

import jax
import jax.numpy as jnp
from jax import lax
from jax.experimental import pallas as pl
from jax.experimental.pallas import tpu as pltpu
from jax.experimental.pallas import tpu_sc as plsc

B, N, D = 16, 2048, 512
TOTAL = B * N
NC, NS = 2, 16
NW = NC * NS
ROWS_PER_W = TOTAL // NW
CHUNK = 64
NCHUNK = ROWS_PER_W // CHUNK


def _gather_kernel(x_hbm, idx_hbm, out_hbm, idx_v, rows_v, sem):
    wid = lax.axis_index("s") * NC + lax.axis_index("c")
    base = pl.multiple_of(wid * ROWS_PER_W, ROWS_PER_W)

    def body(c, _):
        off = pl.multiple_of(base + c * CHUNK, CHUNK)
        pltpu.sync_copy(idx_hbm.at[pl.ds(off, CHUNK)], idx_v)
        pltpu.async_copy(x_hbm.at[idx_v], rows_v, sem).wait()
        pltpu.sync_copy(rows_v, out_hbm.at[pl.ds(off, CHUNK)])
        return ()

    lax.fori_loop(0, NCHUNK, body, (), unroll=False)


@jax.jit
def _expand(x, durations):
    x_flat = x.reshape(TOTAL, D)
    cum_excl = jnp.cumsum(durations.reshape(-1)) - durations.reshape(-1)
    g = jnp.arange(TOTAL, dtype=jnp.int32)
    tok = jnp.searchsorted(cum_excl.astype(jnp.int32), g, side="right") - 1
    tok = jnp.clip(tok, 0, TOTAL - 1).astype(jnp.int32)
    f_idx = (tok % N) + (g // N) * N

    mesh = plsc.VectorSubcoreMesh(core_axis_name="c", subcore_axis_name="s")
    out_flat = pl.kernel(
        _gather_kernel,
        mesh=mesh,
        out_type=jax.ShapeDtypeStruct((TOTAL, D), jnp.float32),
        scratch_types=[
            pltpu.VMEM((CHUNK,), jnp.int32),
            pltpu.VMEM((CHUNK, D), jnp.float32),
            pltpu.SemaphoreType.DMA,
        ],
    )(x_flat, f_idx)

    total_lengths = durations.sum(axis=1)
    mel_mask = jnp.arange(N)[None, :] >= total_lengths[:, None]
    return out_flat.reshape(B, N, D), mel_mask


def kernel(x, durations):
    return _expand(x, durations)


# speedup vs baseline: 1.0235x; 1.0235x over previous
import jax
import jax.numpy as jnp
from jax import lax
from jax.experimental import pallas as pl
from jax.experimental.pallas import tpu as pltpu
from jax.experimental.pallas import tpu_sc as plsc

B, N, D = 16, 2048, 512
TOTAL = B * N
NC, NS = 2, 16
NW = NC * NS
ROWS_PER_W = TOTAL // NW
CHUNK = 64
NCHUNK = ROWS_PER_W // CHUNK
NBUF = 3
LAG = 1


def _gather_kernel(x_hbm, idx_hbm, out_hbm, idx_v, b0, b1, b2,
                   g0, g1, g2, s0, s1, s2):
    wid = lax.axis_index("s") * NC + lax.axis_index("c")
    base = pl.multiple_of(wid * ROWS_PER_W, ROWS_PER_W)
    bufs = (b0, b1, b2)
    gsems = (g0, g1, g2)
    ssems = (s0, s1, s2)

    pltpu.sync_copy(idx_hbm.at[wid], idx_v)

    g_h = {}
    s_h = {}
    for t in range(NCHUNK + LAG):
        if t < NCHUNK:
            bi = t % NBUF
            if t >= NBUF:
                s_h[t - NBUF].wait()
            g_h[t] = pltpu.async_copy(x_hbm.at[idx_v.at[t]], bufs[bi],
                                      gsems[bi])
        tc = t - LAG
        if 0 <= tc < NCHUNK:
            bc = tc % NBUF
            g_h[tc].wait()
            off = pl.multiple_of(base + tc * CHUNK, CHUNK)
            s_h[tc] = pltpu.async_copy(bufs[bc],
                                       out_hbm.at[pl.ds(off, CHUNK)],
                                       ssems[bc])
    for t in range(NCHUNK - NBUF, NCHUNK):
        s_h[t].wait()


@jax.jit
def _expand(x, durations):
    x_flat = x.reshape(TOTAL, D)
    cum_excl = jnp.cumsum(durations.reshape(-1)) - durations.reshape(-1)
    g = jnp.arange(TOTAL, dtype=jnp.int32)
    tok = jnp.searchsorted(cum_excl.astype(jnp.int32), g, side="right") - 1
    tok = jnp.clip(tok, 0, TOTAL - 1).astype(jnp.int32)
    f_idx = ((tok % N) + (g // N) * N).reshape(NW, NCHUNK, CHUNK)

    mesh = plsc.VectorSubcoreMesh(core_axis_name="c", subcore_axis_name="s")
    out_flat = pl.kernel(
        _gather_kernel,
        mesh=mesh,
        out_type=jax.ShapeDtypeStruct((TOTAL, D), jnp.float32),
        scratch_types=(
            [pltpu.VMEM((NCHUNK, CHUNK), jnp.int32)]
            + [pltpu.VMEM((CHUNK, D), jnp.float32)] * NBUF
            + [pltpu.SemaphoreType.DMA] * (2 * NBUF)
        ),
    )(x_flat, f_idx)

    total_lengths = durations.sum(axis=1)
    mel_mask = jnp.arange(N)[None, :] >= total_lengths[:, None]
    return out_flat.reshape(B, N, D), mel_mask


def kernel(x, durations):
    return _expand(x, durations)


# speedup vs baseline: 8.2637x; 8.0742x over previous
import jax
import jax.numpy as jnp
from jax import lax
from jax.experimental import pallas as pl
from jax.experimental.pallas import tpu as pltpu
from jax.experimental.pallas import tpu_sc as plsc

B, N, D = 16, 2048, 512
TOTAL = B * N
NC, NS = 2, 16
NW = NC * NS
ROWS_PER_W = TOTAL // NW
CHUNK = 32
NCHUNK = ROWS_PER_W // CHUNK
NBUF = 4
LAG = 2
LANES = 16


def _prep_kernel(dur_ref, cum_ref, mask_ref):
    dur = dur_ref[:]
    lane = lax.broadcasted_iota(jnp.int32, (B, N), 1)
    sub = lax.broadcasted_iota(jnp.int32, (B, 1), 0)
    c = dur
    k = 1
    while k < N:
        c = c + jnp.where(lane >= k, pltpu.roll(c, k, axis=1), 0)
        k *= 2
    totals = c[:, N - 1:N]
    p = totals
    k = 1
    while k < B:
        p = p + jnp.where(sub >= k, pltpu.roll(p, k, axis=0), 0)
        k *= 2
    cum_ref[:] = (p - totals) + (c - dur)
    mask_ref[:] = lane >= totals


def _gather_kernel(x_hbm, cum_hbm, out_hbm, cum_v, idx_v,
                   b0, b1, b2, b3, g0, g1, g2, g3, s0, s1, s2, s3):
    wid = lax.axis_index("s") * NC + lax.axis_index("c")
    base = wid * ROWS_PER_W
    bufs = (b0, b1, b2, b3)
    gsems = (g0, g1, g2, g3)
    ssems = (s0, s1, s2, s3)

    pltpu.sync_copy(cum_hbm, cum_v)
    lane = lax.iota(jnp.int32, LANES)

    def search_body(v, _):
        g = base + v * LANES + lane
        res = jnp.zeros((LANES,), jnp.int32)
        for s in (2 ** e for e in reversed(range(16))):
            cand = res + s
            val = plsc.load_gather(cum_v, [jnp.minimum(cand - 1, TOTAL - 1)])
            ok = (val <= g) & (cand <= TOTAL)
            res = jnp.where(ok, cand, res)
        tok = res - 1
        idx_v[pl.ds(v * LANES, LANES)] = (tok & (N - 1)) + (g - (g & (N - 1)))
        return ()

    lax.fori_loop(0, ROWS_PER_W // LANES, search_body, ())

    g_h = {}
    s_h = {}
    for t in range(NCHUNK + LAG):
        if t < NCHUNK:
            bi = t % NBUF
            if t >= NBUF:
                s_h[t - NBUF].wait()
            g_h[t] = pltpu.async_copy(
                x_hbm.at[idx_v.at[pl.ds(t * CHUNK, CHUNK)]], bufs[bi],
                gsems[bi])
        tc = t - LAG
        if 0 <= tc < NCHUNK:
            bc = tc % NBUF
            g_h[tc].wait()
            off = pl.multiple_of(base + tc * CHUNK, CHUNK)
            s_h[tc] = pltpu.async_copy(bufs[bc],
                                       out_hbm.at[pl.ds(off, CHUNK)],
                                       ssems[bc])
    for t in range(NCHUNK - NBUF, NCHUNK):
        s_h[t].wait()


@jax.jit
def _expand(x, durations):
    x_flat = x.reshape(TOTAL, D)

    cum, mel_mask = pl.pallas_call(
        _prep_kernel,
        out_shape=[
            jax.ShapeDtypeStruct((B, N), jnp.int32),
            jax.ShapeDtypeStruct((B, N), jnp.bool_),
        ],
    )(durations)

    mesh = plsc.VectorSubcoreMesh(core_axis_name="c", subcore_axis_name="s")
    out_flat = pl.kernel(
        _gather_kernel,
        mesh=mesh,
        out_type=jax.ShapeDtypeStruct((TOTAL, D), jnp.float32),
        compiler_params=pltpu.CompilerParams(needs_layout_passes=False),
        scratch_types=(
            [pltpu.VMEM((TOTAL,), jnp.int32),
             pltpu.VMEM((ROWS_PER_W,), jnp.int32)]
            + [pltpu.VMEM((CHUNK, D), jnp.float32)] * NBUF
            + [pltpu.SemaphoreType.DMA] * (2 * NBUF)
        ),
    )(x_flat, cum.reshape(TOTAL))

    return out_flat.reshape(B, N, D), mel_mask


def kernel(x, durations):
    return _expand(x, durations)


# speedup vs baseline: 8.3262x; 1.0076x over previous
import jax
import jax.numpy as jnp
from jax import lax
from jax.experimental import pallas as pl
from jax.experimental.pallas import tpu as pltpu
from jax.experimental.pallas import tpu_sc as plsc

B, N, D = 16, 2048, 512
TOTAL = B * N
NC, NS = 2, 16
NW = NC * NS
ROWS_PER_W = TOTAL // NW
CHUNK = 32
NCHUNK = ROWS_PER_W // CHUNK
NBUF = 5
LAG = 2
LANES = 16


def _prep_kernel(dur_ref, cum_ref, mask_ref):
    dur = dur_ref[:]
    lane = lax.broadcasted_iota(jnp.int32, (B, N), 1)
    sub = lax.broadcasted_iota(jnp.int32, (B, 1), 0)
    c = dur
    k = 1
    while k < N:
        c = c + jnp.where(lane >= k, pltpu.roll(c, k, axis=1), 0)
        k *= 2
    totals = c[:, N - 1:N]
    p = totals
    k = 1
    while k < B:
        p = p + jnp.where(sub >= k, pltpu.roll(p, k, axis=0), 0)
        k *= 2
    cum_ref[:] = (p - totals) + (c - dur)
    mask_ref[:] = lane >= totals


def _gather_kernel(x_hbm, cum_hbm, out_hbm, cum_v, idx_v,
                   b0, b1, b2, b3, b4, g0, g1, g2, g3, g4,
                   s0, s1, s2, s3, s4):
    wid = lax.axis_index("s") * NC + lax.axis_index("c")
    base = wid * ROWS_PER_W
    bufs = (b0, b1, b2, b3, b4)
    gsems = (g0, g1, g2, g3, g4)
    ssems = (s0, s1, s2, s3, s4)

    pltpu.sync_copy(cum_hbm, cum_v)
    lane = lax.iota(jnp.int32, LANES)

    def _search_one(g):
        res = jnp.zeros((LANES,), jnp.int32)
        for s in (2 ** e for e in reversed(range(16))):
            cand = res + s
            val = plsc.load_gather(cum_v, [jnp.minimum(cand - 1, TOTAL - 1)])
            ok = (val <= g) & (cand <= TOTAL)
            res = jnp.where(ok, cand, res)
        tok = res - 1
        return (tok & (N - 1)) + (g - (g & (N - 1)))

    def search_body(v, _):
        g0v = base + v * (2 * LANES) + lane
        g1v = g0v + LANES
        idx_v[pl.ds(v * (2 * LANES), LANES)] = _search_one(g0v)
        idx_v[pl.ds(v * (2 * LANES) + LANES, LANES)] = _search_one(g1v)
        return ()

    lax.fori_loop(0, ROWS_PER_W // (2 * LANES), search_body, ())

    g_h = {}
    s_h = {}
    for t in range(NCHUNK + LAG):
        if t < NCHUNK:
            bi = t % NBUF
            if t >= NBUF:
                s_h[t - NBUF].wait()
            g_h[t] = pltpu.async_copy(
                x_hbm.at[idx_v.at[pl.ds(t * CHUNK, CHUNK)]], bufs[bi],
                gsems[bi])
        tc = t - LAG
        if 0 <= tc < NCHUNK:
            bc = tc % NBUF
            g_h[tc].wait()
            off = pl.multiple_of(base + tc * CHUNK, CHUNK)
            s_h[tc] = pltpu.async_copy(bufs[bc],
                                       out_hbm.at[pl.ds(off, CHUNK)],
                                       ssems[bc])
    for t in range(NCHUNK - NBUF, NCHUNK):
        s_h[t].wait()


@jax.jit
def _expand(x, durations):
    x_flat = x.reshape(TOTAL, D)

    cum, mel_mask = pl.pallas_call(
        _prep_kernel,
        out_shape=[
            jax.ShapeDtypeStruct((B, N), jnp.int32),
            jax.ShapeDtypeStruct((B, N), jnp.bool_),
        ],
    )(durations)

    mesh = plsc.VectorSubcoreMesh(core_axis_name="c", subcore_axis_name="s")
    out_flat = pl.kernel(
        _gather_kernel,
        mesh=mesh,
        out_type=jax.ShapeDtypeStruct((TOTAL, D), jnp.float32),
        compiler_params=pltpu.CompilerParams(needs_layout_passes=False),
        scratch_types=(
            [pltpu.VMEM((TOTAL,), jnp.int32),
             pltpu.VMEM((ROWS_PER_W,), jnp.int32)]
            + [pltpu.VMEM((CHUNK, D), jnp.float32)] * NBUF
            + [pltpu.SemaphoreType.DMA] * (2 * NBUF)
        ),
    )(x_flat, cum.reshape(TOTAL))

    return out_flat.reshape(B, N, D), mel_mask


def kernel(x, durations):
    return _expand(x, durations)


# speedup vs baseline: 8.8885x; 1.0675x over previous
import jax
import jax.numpy as jnp
from jax import lax
from jax.experimental import pallas as pl
from jax.experimental.pallas import tpu as pltpu
from jax.experimental.pallas import tpu_sc as plsc

B, N, D = 16, 2048, 512
TOTAL = B * N
NC, NS = 2, 16
NW = NC * NS
ROWS_PER_W = TOTAL // NW
CHUNK = 32
NCHUNK = ROWS_PER_W // CHUNK
NBUF = 5
LAG = 2
LANES = 16


def _prep_kernel(dur_ref, cum_ref, mask_ref):
    dur = dur_ref[:]
    lane = lax.broadcasted_iota(jnp.int32, (B, N), 1)
    sub = lax.broadcasted_iota(jnp.int32, (B, 1), 0)
    c = dur
    k = 1
    while k < N:
        c = c + jnp.where(lane >= k, pltpu.roll(c, k, axis=1), 0)
        k *= 2
    totals = c[:, N - 1:N]
    p = totals
    k = 1
    while k < B:
        p = p + jnp.where(sub >= k, pltpu.roll(p, k, axis=0), 0)
        k *= 2
    cum_ref[:] = (p - totals) + (c - dur)
    mask_ref[:] = lane >= totals


def _gather_kernel(x_hbm, cum_hbm, out_hbm, cum_v, idx_v,
                   b0, b1, b2, b3, b4, g0, g1, g2, g3, g4,
                   s0, s1, s2, s3, s4):
    wid = lax.axis_index("s") * NC + lax.axis_index("c")
    base = wid * ROWS_PER_W
    bufs = (b0, b1, b2, b3, b4)
    gsems = (g0, g1, g2, g3, g4)
    ssems = (s0, s1, s2, s3, s4)

    pltpu.sync_copy(cum_hbm, cum_v)
    lane = lax.iota(jnp.int32, LANES)

    def _search_one(g):
        res = jnp.zeros((LANES,), jnp.int32)
        for s in (2 ** e for e in reversed(range(16))):
            cand = res + s
            val = plsc.load_gather(cum_v, [jnp.minimum(cand - 1, TOTAL - 1)])
            ok = (val <= g) & (cand <= TOTAL)
            res = jnp.where(ok, cand, res)
        tok = res - 1
        return (tok & (N - 1)) + (g - (g & (N - 1)))

    def search_body(v, _):
        g0v = base + v * (2 * LANES) + lane
        g1v = g0v + LANES
        idx_v[pl.ds(v * (2 * LANES), LANES)] = _search_one(g0v)
        idx_v[pl.ds(v * (2 * LANES) + LANES, LANES)] = _search_one(g1v)
        return ()

    GROUPS = 4
    CH_PER_G = NCHUNK // GROUPS
    PAIRS_PER_G = (ROWS_PER_W // (2 * LANES)) // GROUPS

    g_h = {}
    s_h = {}
    for t in range(NCHUNK + LAG):
        if t < NCHUNK and t % CH_PER_G == 0:
            grp = t // CH_PER_G
            lax.fori_loop(grp * PAIRS_PER_G, (grp + 1) * PAIRS_PER_G,
                          search_body, ())
        if t < NCHUNK:
            bi = t % NBUF
            if t >= NBUF:
                s_h[t - NBUF].wait()
            g_h[t] = pltpu.async_copy(
                x_hbm.at[idx_v.at[pl.ds(t * CHUNK, CHUNK)]], bufs[bi],
                gsems[bi])
        tc = t - LAG
        if 0 <= tc < NCHUNK:
            bc = tc % NBUF
            g_h[tc].wait()
            off = pl.multiple_of(base + tc * CHUNK, CHUNK)
            s_h[tc] = pltpu.async_copy(bufs[bc],
                                       out_hbm.at[pl.ds(off, CHUNK)],
                                       ssems[bc])
    for t in range(NCHUNK - NBUF, NCHUNK):
        s_h[t].wait()


@jax.jit
def _expand(x, durations):
    x_flat = x.reshape(TOTAL, D)

    cum, mel_mask = pl.pallas_call(
        _prep_kernel,
        out_shape=[
            jax.ShapeDtypeStruct((B, N), jnp.int32),
            jax.ShapeDtypeStruct((B, N), jnp.bool_),
        ],
    )(durations)

    mesh = plsc.VectorSubcoreMesh(core_axis_name="c", subcore_axis_name="s")
    out_flat = pl.kernel(
        _gather_kernel,
        mesh=mesh,
        out_type=jax.ShapeDtypeStruct((TOTAL, D), jnp.float32),
        compiler_params=pltpu.CompilerParams(needs_layout_passes=False),
        scratch_types=(
            [pltpu.VMEM((TOTAL,), jnp.int32),
             pltpu.VMEM((ROWS_PER_W,), jnp.int32)]
            + [pltpu.VMEM((CHUNK, D), jnp.float32)] * NBUF
            + [pltpu.SemaphoreType.DMA] * (2 * NBUF)
        ),
    )(x_flat, cum.reshape(TOTAL))

    return out_flat.reshape(B, N, D), mel_mask


def kernel(x, durations):
    return _expand(x, durations)
